# baseline (speedup 1.0000x reference)
import functools

import jax
import jax.numpy as jnp
from jax import lax
from jax.experimental import pallas as pl
from jax.experimental.pallas import tpu as pltpu
from jax.experimental.pallas import tpu_sc as plsc

NW = 32
LANES = 16

N1_ROWS = 50000
K = 32
INV0 = 100000
TW1 = 784
SUB0_W = 12500
GROUPS_W = 98
CH = 14
NCHUNK = 7
ROWS_CHUNK = CH * LANES
WCHUNK = ROWS_CHUNK * K // 128
PAD_ROWS = NW * GROUPS_W * LANES

INV1 = 50000
TW2 = 392
M2 = 8192
PW = 256
G2 = PW // LANES

_mesh = plsc.VectorSubcoreMesh(core_axis_name="c", subcore_axis_name="s")
_sc_params = pltpu.CompilerParams(needs_layout_passes=False,
                                  use_tc_tiling_on_sc=False)


def _worker_id():
    return lax.axis_index("s") * 2 + lax.axis_index("c")


@functools.partial(
    pl.kernel,
    mesh=_mesh,
    out_type=jax.ShapeDtypeStruct((PAD_ROWS,), jnp.float32),
    scratch_types=[
        pltpu.VMEM((TW1, 128), jnp.float32),
        pltpu.VMEM((3 * ROWS_CHUNK, K), jnp.int32),
        pltpu.VMEM((2, ROWS_CHUNK), jnp.float32),
        pltpu.SemaphoreType.DMA,
        pltpu.SemaphoreType.DMA,
        pltpu.SemaphoreType.DMA,
        pltpu.SemaphoreType.DMA,
        pltpu.SemaphoreType.DMA,
        pltpu.SemaphoreType.DMA,
    ],
    compiler_params=_sc_params,
)
def _lvl1(pyr0_hbm, sub0_hbm, out_hbm, table_v, slab_v, outc_v,
          sem_t, sem_s0, sem_s1, sem_s2, sem_o0, sem_o1):
    sem_s = (sem_s0, sem_s1, sem_s2)
    sem_o = (sem_o0, sem_o1)
    w = _worker_id()
    lane = lax.broadcasted_iota(jnp.int32, (LANES,), 0)

    def chunk_base(c):
        r0 = (w * GROUPS_W + c * CH) * LANES
        r0c = jnp.minimum(r0, N1_ROWS - ROWS_CHUNK)
        return r0, r0c

    def fire_slab(c):
        slot = c % 3
        _, r0c = chunk_base(c)
        for i in range(CH):
            vec = r0c + i * LANES + lane
            pltpu.async_copy(
                sub0_hbm.at[vec],
                slab_v.at[pl.ds(slot * ROWS_CHUNK + i * LANES, LANES)],
                sem_s[slot])

    for c in range(3):
        fire_slab(c)
    for i in range(TW1 // LANES):
        pltpu.async_copy(pyr0_hbm.at[lane + i * LANES],
                         table_v.at[pl.ds(i * LANES, LANES)], sem_t)
    pltpu.make_async_copy(pyr0_hbm.at[pl.ds(0, TW1)], table_v, sem_t).wait()

    for c in range(NCHUNK):
        slot = c % 3
        oslot = c % 2
        r0, r0c = chunk_base(c)
        pltpu.make_async_copy(sub0_hbm.at[pl.ds(0, ROWS_CHUNK)],
                              slab_v.at[pl.ds(slot * ROWS_CHUNK, ROWS_CHUNK)],
                              sem_s[slot]).wait()
        if c >= 2:
            pltpu.make_async_copy(outc_v.at[oslot],
                                  out_hbm.at[pl.ds(0, ROWS_CHUNK)],
                                  sem_o[oslot]).wait()

        def group_body(j, carry, *, _slot=slot, _oslot=oslot,
                       _r0=r0, _r0c=r0c):
            base_row = jnp.minimum(_r0 - _r0c + j * LANES,
                                   ROWS_CHUNK - LANES)
            lr = _slot * ROWS_CHUNK + base_row + lane
            acc = jnp.zeros((LANES,), jnp.float32)
            cnt = jnp.zeros((LANES,), jnp.float32)
            for k in range(K):
                col = jnp.full((LANES,), k, jnp.int32)
                idx = plsc.load_gather(slab_v, [lr, col])
                valid = idx < INV0
                safe = jnp.where(valid, idx, 0)
                vals = plsc.load_gather(table_v, [safe >> 7, safe & 127])
                vf = valid.astype(jnp.float32)
                acc = acc + vals * vf
                cnt = cnt + vf
            outc_v[_oslot, pl.ds(j * LANES, LANES)] = jnp.clip(
                acc / cnt, 0.0, 1.0)
            return carry

        lax.fori_loop(0, CH, group_body, 0)
        if c + 3 < NCHUNK:
            fire_slab(c + 3)
        pltpu.async_copy(outc_v.at[oslot], out_hbm.at[pl.ds(r0, ROWS_CHUNK)],
                         sem_o[oslot])

    for oslot in (1, 0):
        pltpu.make_async_copy(outc_v.at[oslot],
                              out_hbm.at[pl.ds(0, ROWS_CHUNK)],
                              sem_o[oslot]).wait()


@functools.partial(
    pl.kernel,
    mesh=_mesh,
    out_type=jax.ShapeDtypeStruct((M2,), jnp.float32),
    scratch_types=[
        pltpu.VMEM((TW2 + 8, 128), jnp.float32),
        pltpu.VMEM((2, 128), jnp.int32),
        pltpu.VMEM((PW, K), jnp.int32),
        pltpu.VMEM((PW,), jnp.float32),
        pltpu.SemaphoreType.DMA,
        pltpu.SemaphoreType.DMA,
    ],
    compiler_params=_sc_params,
)
def _lvl2(pyr1_hbm, sub1_hbm, rows_hbm, out_hbm,
          table_v, rows_v, slab_v, out_v, sem_t, sem_s):
    w = _worker_id()
    lane = lax.broadcasted_iota(jnp.int32, (LANES,), 0)
    for i in range((TW2 + 8) // LANES):
        vec = jnp.minimum(i * LANES + lane, TW2 - 1)
        pltpu.async_copy(pyr1_hbm.at[vec],
                         table_v.at[pl.ds(i * LANES, LANES)], sem_t)
    pltpu.sync_copy(rows_hbm.at[pl.ds(w * 2, 2)], rows_v)
    for j in range(2):
        pltpu.async_copy(sub1_hbm.at[rows_v.at[j]],
                         slab_v.at[pl.ds(j * 128, 128)], sem_s)
    pltpu.make_async_copy(sub1_hbm.at[pl.ds(0, PW)], slab_v, sem_s).wait()
    pltpu.make_async_copy(pyr1_hbm.at[pl.ds(0, 384)],
                          table_v.at[pl.ds(0, 384)], sem_t).wait()
    pltpu.make_async_copy(pyr1_hbm.at[pl.ds(0, 16)],
                          table_v.at[pl.ds(384, 16)], sem_t).wait()

    def group_body(g, carry):
        s = g * LANES + lane
        acc = jnp.zeros((LANES,), jnp.float32)
        cnt = jnp.zeros((LANES,), jnp.float32)
        for k in range(K):
            col = jnp.full((LANES,), k, jnp.int32)
            idx = plsc.load_gather(slab_v, [s, col])
            valid = idx < INV1
            safe = jnp.where(valid, idx, 0)
            vals = plsc.load_gather(table_v, [safe >> 7, safe & 127])
            vf = valid.astype(jnp.float32)
            acc = acc + vals * vf
            cnt = cnt + vf
        out_v[pl.ds(g * LANES, LANES)] = jnp.clip(acc / cnt, 0.0, 1.0)
        return carry

    lax.fori_loop(0, G2, group_body, 0)
    pltpu.sync_copy(out_v, out_hbm.at[pl.ds(w * PW, PW)])


def _bce_body(gt_ref, lg_ref, out_ref):
    gt = gt_ref[...]
    lg = lg_ref[...]
    t = jnp.maximum(lg, 0.0) - lg * gt + jnp.log1p(jnp.exp(-jnp.abs(lg)))
    out_ref[0, 0] = jnp.sum(t) / float(M2)


def kernel(ref_overlap, src_overlap, ref_overlap_pred, src_overlap_pred,
           lengths_0, lengths_1, lengths_2, subsampling_0, subsampling_1,
           ref_indices, src_indices):
    pad = jnp.zeros((TW1 * 128 - 2 * ref_overlap.shape[0],), jnp.float32)
    pyr0w = jnp.reshape(
        jnp.concatenate([ref_overlap.astype(jnp.float32),
                         src_overlap.astype(jnp.float32), pad], axis=0),
        (TW1, 128))
    pyr1 = _lvl1(pyr0w, subsampling_0)

    rows = jnp.concatenate([ref_indices, src_indices + lengths_2[0]],
                           axis=0).astype(jnp.int32)
    rows2d = jnp.reshape(rows, (NW * 2, 128))
    pyr1w = jnp.reshape(pyr1, (TW2, 128))
    gt = _lvl2(pyr1w, subsampling_1, rows2d)

    logits = jnp.concatenate([ref_overlap_pred, src_overlap_pred], axis=-2)[:, 0]
    lg2d = jnp.reshape(logits.astype(jnp.float32), (64, 128))
    gt2d = jnp.reshape(gt, (64, 128))
    loss = pl.pallas_call(
        _bce_body,
        out_shape=jax.ShapeDtypeStruct((1, 1), jnp.float32),
        out_specs=pl.BlockSpec(memory_space=pltpu.SMEM),
    )(gt2d, lg2d)
    return loss[0, 0]


# speedup vs baseline: 1.1263x; 1.1263x over previous
import functools

import jax
import jax.numpy as jnp
from jax import lax
from jax.experimental import pallas as pl
from jax.experimental.pallas import tpu as pltpu
from jax.experimental.pallas import tpu_sc as plsc

NW = 32
LANES = 16

N1_ROWS = 50000
K = 32
INV0 = 100000
TW1 = 784
SUB0_W = 12500
GROUPS_W = 98
CH = 14
NCHUNK = 7
ROWS_CHUNK = CH * LANES
WCHUNK = ROWS_CHUNK * K // 128
PAD_ROWS = NW * GROUPS_W * LANES

INV1 = 50000
TW2 = 392
M2 = 8192
PW = 256
G2 = PW // LANES

_mesh = plsc.VectorSubcoreMesh(core_axis_name="c", subcore_axis_name="s")
_sc_params = pltpu.CompilerParams(needs_layout_passes=False,
                                  use_tc_tiling_on_sc=False)


def _worker_id():
    return lax.axis_index("c") * 16 + lax.axis_index("s")


@functools.partial(
    pl.kernel,
    mesh=_mesh,
    out_type=jax.ShapeDtypeStruct((PAD_ROWS,), jnp.float32),
    scratch_types=[
        pltpu.VMEM((TW1, 128), jnp.float32),
        pltpu.VMEM_SHARED((TW1, 128), jnp.float32),
        pltpu.VMEM((3 * ROWS_CHUNK, K), jnp.int32),
        pltpu.VMEM((2, ROWS_CHUNK), jnp.float32),
        pltpu.SemaphoreType.DMA,
        pltpu.SemaphoreType.DMA,
        pltpu.SemaphoreType.DMA,
        pltpu.SemaphoreType.DMA,
        pltpu.SemaphoreType.DMA,
        pltpu.SemaphoreType.DMA,
    ],
    compiler_params=_sc_params,
)
def _lvl1(pyr0_hbm, sub0_hbm, out_hbm, table_v, table_sh, slab_v, outc_v,
          sem_t, sem_s0, sem_s1, sem_s2, sem_o0, sem_o1):
    sem_s = (sem_s0, sem_s1, sem_s2)
    sem_o = (sem_o0, sem_o1)
    w = _worker_id()
    lane = lax.broadcasted_iota(jnp.int32, (LANES,), 0)

    def chunk_base(c):
        r0 = (w * GROUPS_W + c * CH) * LANES
        r0c = jnp.minimum(r0, N1_ROWS - ROWS_CHUNK)
        return r0, r0c

    def fire_slab(c):
        slot = c % 3
        _, r0c = chunk_base(c)
        for i in range(CH):
            vec = r0c + i * LANES + lane
            pltpu.async_copy(
                sub0_hbm.at[vec],
                slab_v.at[pl.ds(slot * ROWS_CHUNK + i * LANES, LANES)],
                sem_s[slot])

    for c in range(3):
        fire_slab(c)

    @pl.when(lax.axis_index("s") == 0)
    def _():
        pltpu.sync_copy(pyr0_hbm, table_sh)

    plsc.subcore_barrier()
    pltpu.sync_copy(table_sh, table_v)

    for c in range(NCHUNK):
        slot = c % 3
        oslot = c % 2
        r0, r0c = chunk_base(c)
        pltpu.make_async_copy(sub0_hbm.at[pl.ds(0, ROWS_CHUNK)],
                              slab_v.at[pl.ds(slot * ROWS_CHUNK, ROWS_CHUNK)],
                              sem_s[slot]).wait()
        if c >= 2:
            pltpu.make_async_copy(outc_v.at[oslot],
                                  out_hbm.at[pl.ds(0, ROWS_CHUNK)],
                                  sem_o[oslot]).wait()

        def group_body(j, carry, *, _slot=slot, _oslot=oslot,
                       _r0=r0, _r0c=r0c):
            base_row = jnp.minimum(_r0 - _r0c + j * LANES,
                                   ROWS_CHUNK - LANES)
            lr = _slot * ROWS_CHUNK + base_row + lane
            acc = jnp.zeros((LANES,), jnp.float32)
            cnt = jnp.zeros((LANES,), jnp.float32)
            for k in range(K):
                col = jnp.full((LANES,), k, jnp.int32)
                idx = plsc.load_gather(slab_v, [lr, col])
                valid = idx < INV0
                safe = jnp.where(valid, idx, 0)
                vals = plsc.load_gather(table_v, [safe >> 7, safe & 127])
                vf = valid.astype(jnp.float32)
                acc = acc + vals * vf
                cnt = cnt + vf
            outc_v[_oslot, pl.ds(j * LANES, LANES)] = jnp.clip(
                acc / cnt, 0.0, 1.0)
            return carry

        lax.fori_loop(0, CH, group_body, 0)
        if c + 3 < NCHUNK:
            fire_slab(c + 3)
        pltpu.async_copy(outc_v.at[oslot], out_hbm.at[pl.ds(r0, ROWS_CHUNK)],
                         sem_o[oslot])

    for oslot in (1, 0):
        pltpu.make_async_copy(outc_v.at[oslot],
                              out_hbm.at[pl.ds(0, ROWS_CHUNK)],
                              sem_o[oslot]).wait()


@functools.partial(
    pl.kernel,
    mesh=_mesh,
    out_type=jax.ShapeDtypeStruct((M2,), jnp.float32),
    scratch_types=[
        pltpu.VMEM((TW2 + 8, 128), jnp.float32),
        pltpu.VMEM((2, 128), jnp.int32),
        pltpu.VMEM((PW, K), jnp.int32),
        pltpu.VMEM((PW,), jnp.float32),
        pltpu.SemaphoreType.DMA,
        pltpu.SemaphoreType.DMA,
    ],
    compiler_params=_sc_params,
)
def _lvl2(pyr1_hbm, sub1_hbm, rows_hbm, out_hbm,
          table_v, rows_v, slab_v, out_v, sem_t, sem_s):
    w = _worker_id()
    lane = lax.broadcasted_iota(jnp.int32, (LANES,), 0)
    for i in range((TW2 + 8) // LANES):
        vec = jnp.minimum(i * LANES + lane, TW2 - 1)
        pltpu.async_copy(pyr1_hbm.at[vec],
                         table_v.at[pl.ds(i * LANES, LANES)], sem_t)
    pltpu.sync_copy(rows_hbm.at[pl.ds(w * 2, 2)], rows_v)
    for j in range(2):
        pltpu.async_copy(sub1_hbm.at[rows_v.at[j]],
                         slab_v.at[pl.ds(j * 128, 128)], sem_s)
    pltpu.make_async_copy(sub1_hbm.at[pl.ds(0, PW)], slab_v, sem_s).wait()
    pltpu.make_async_copy(pyr1_hbm.at[pl.ds(0, 384)],
                          table_v.at[pl.ds(0, 384)], sem_t).wait()
    pltpu.make_async_copy(pyr1_hbm.at[pl.ds(0, 16)],
                          table_v.at[pl.ds(384, 16)], sem_t).wait()

    def group_body(g, carry):
        s = g * LANES + lane
        acc = jnp.zeros((LANES,), jnp.float32)
        cnt = jnp.zeros((LANES,), jnp.float32)
        for k in range(K):
            col = jnp.full((LANES,), k, jnp.int32)
            idx = plsc.load_gather(slab_v, [s, col])
            valid = idx < INV1
            safe = jnp.where(valid, idx, 0)
            vals = plsc.load_gather(table_v, [safe >> 7, safe & 127])
            vf = valid.astype(jnp.float32)
            acc = acc + vals * vf
            cnt = cnt + vf
        out_v[pl.ds(g * LANES, LANES)] = jnp.clip(acc / cnt, 0.0, 1.0)
        return carry

    lax.fori_loop(0, G2, group_body, 0)
    pltpu.sync_copy(out_v, out_hbm.at[pl.ds(w * PW, PW)])


def _bce_body(gt_ref, lg_ref, out_ref):
    gt = gt_ref[...]
    lg = lg_ref[...]
    t = jnp.maximum(lg, 0.0) - lg * gt + jnp.log1p(jnp.exp(-jnp.abs(lg)))
    out_ref[0, 0] = jnp.sum(t) / float(M2)


def kernel(ref_overlap, src_overlap, ref_overlap_pred, src_overlap_pred,
           lengths_0, lengths_1, lengths_2, subsampling_0, subsampling_1,
           ref_indices, src_indices):
    pad = jnp.zeros((TW1 * 128 - 2 * ref_overlap.shape[0],), jnp.float32)
    pyr0w = jnp.reshape(
        jnp.concatenate([ref_overlap.astype(jnp.float32),
                         src_overlap.astype(jnp.float32), pad], axis=0),
        (TW1, 128))
    pyr1 = _lvl1(pyr0w, subsampling_0)

    rows = jnp.concatenate([ref_indices, src_indices + lengths_2[0]],
                           axis=0).astype(jnp.int32)
    rows2d = jnp.reshape(rows, (NW * 2, 128))
    pyr1w = jnp.reshape(pyr1, (TW2, 128))
    gt = _lvl2(pyr1w, subsampling_1, rows2d)

    logits = jnp.concatenate([ref_overlap_pred, src_overlap_pred], axis=-2)[:, 0]
    lg2d = jnp.reshape(logits.astype(jnp.float32), (64, 128))
    gt2d = jnp.reshape(gt, (64, 128))
    loss = pl.pallas_call(
        _bce_body,
        out_shape=jax.ShapeDtypeStruct((1, 1), jnp.float32),
        out_specs=pl.BlockSpec(memory_space=pltpu.SMEM),
    )(gt2d, lg2d)
    return loss[0, 0]
